# Initial kernel scaffold; baseline (speedup 1.0000x reference)
import jax
import jax.numpy as jnp
from jax.experimental import pallas as pl


def kernel(features):
    raise NotImplementedError("write your pallas kernel here")


# speedup vs baseline: 1.0767x; 1.0767x over previous
import functools

import jax
import jax.numpy as jnp
from jax import lax
from jax.experimental import pallas as pl
from jax.experimental.pallas import tpu as pltpu
from jax.experimental.pallas import tpu_sc as plsc

N_ROWS = 100000
N_COLS = 512
BLOCK_ROWS = 2000
NUM_BLOCKS = N_ROWS // BLOCK_ROWS

N_TILES = 32
BINS_PER_TILE = 3136
N_BINS_PAD = N_TILES * BINS_PER_TILE
LANES = 16
INV_TOTAL = 1.0 / N_COLS


def _maxpool_body(x_ref, max_out, idx_out):
    i = pl.program_id(0)
    x = x_ref[...]
    bmax = jnp.max(x, axis=0, keepdims=True)
    rows = lax.broadcasted_iota(jnp.int32, x.shape, 0) + i * BLOCK_ROWS
    bidx = jnp.min(
        jnp.where(x == bmax, rows, jnp.int32(N_ROWS)), axis=0, keepdims=True
    )

    @pl.when(i == 0)
    def _():
        max_out[...] = bmax
        idx_out[...] = bidx

    @pl.when(i > 0)
    def _():
        better = bmax > max_out[...]
        idx_out[...] = jnp.where(better, bidx, idx_out[...])
        max_out[...] = jnp.where(better, bmax, max_out[...])


_maxpool = pl.pallas_call(
    _maxpool_body,
    grid=(NUM_BLOCKS,),
    in_specs=[pl.BlockSpec((BLOCK_ROWS, N_COLS), lambda i: (i, 0))],
    out_specs=[
        pl.BlockSpec((1, N_COLS), lambda i: (0, 0)),
        pl.BlockSpec((1, N_COLS), lambda i: (0, 0)),
    ],
    out_shape=[
        jax.ShapeDtypeStruct((1, N_COLS), jnp.float32),
        jax.ShapeDtypeStruct((1, N_COLS), jnp.int32),
    ],
)


N_SUBCORES = 16
IDX_PER_TILE = N_COLS // N_SUBCORES
ZERO_PER_TILE = N_BINS_PAD // N_SUBCORES
BINS_PER_CORE = N_BINS_PAD // 2


def _hist_body(idx_hbm, out_hbm, idx_row_v, vals_v, zeros_v, hist_sh):
    cid = lax.axis_index("c")
    sid = lax.axis_index("s")
    zeros16 = jnp.zeros((LANES,), jnp.float32)
    for i in range(ZERO_PER_TILE // LANES):
        zeros_v[pl.ds(i * LANES, LANES)] = zeros16
    vals16 = jnp.full((LANES,), INV_TOTAL, jnp.float32)
    for i in range(IDX_PER_TILE // LANES):
        vals_v[pl.ds(i * LANES, LANES)] = vals16
    pltpu.sync_copy(idx_hbm.at[sid], idx_row_v)
    pltpu.sync_copy(zeros_v, hist_sh.at[pl.ds(sid * ZERO_PER_TILE, ZERO_PER_TILE)])
    plsc.subcore_barrier()
    pltpu.sync_copy(vals_v, hist_sh.at[idx_row_v], add=True)
    plsc.subcore_barrier()
    goff = cid * BINS_PER_CORE + sid * BINS_PER_TILE
    out_v = zeros_v.at[pl.ds(0, BINS_PER_TILE)]
    pltpu.sync_copy(hist_sh.at[pl.ds(goff, BINS_PER_TILE)], out_v)
    pltpu.sync_copy(out_v, out_hbm.at[pl.ds(goff, BINS_PER_TILE)])


@functools.cache
def _hist():
    return functools.partial(
        pl.kernel,
        mesh=plsc.VectorSubcoreMesh(core_axis_name="c", subcore_axis_name="s"),
        out_type=jax.ShapeDtypeStruct((N_BINS_PAD,), jnp.float32),
        scratch_types=[
            pltpu.VMEM((IDX_PER_TILE,), jnp.int32),
            pltpu.VMEM((IDX_PER_TILE,), jnp.float32),
            pltpu.VMEM((ZERO_PER_TILE,), jnp.float32),
            pltpu.VMEM_SHARED((N_BINS_PAD,), jnp.float32),
        ],
    )(_hist_body)


@jax.jit
def kernel(features):
    pooled, indices = _maxpool(features)
    hist = _hist()(indices.reshape(N_SUBCORES, IDX_PER_TILE))
    attention_weights = hist[:N_ROWS].reshape(1, N_ROWS)
    return (attention_weights, pooled)


# speedup vs baseline: 1.2482x; 1.1593x over previous
import functools

import jax
import jax.numpy as jnp
from jax import lax
from jax.experimental import pallas as pl
from jax.experimental.pallas import tpu as pltpu
from jax.experimental.pallas import tpu_sc as plsc

N_ROWS = 100000
N_COLS = 512
BLOCK_ROWS = 5000
NUM_BLOCKS = N_ROWS // BLOCK_ROWS

N_TILES = 32
BINS_PER_TILE = 3136
N_BINS_PAD = N_TILES * BINS_PER_TILE
LANES = 16
INV_TOTAL = 1.0 / N_COLS


def _maxpool_body(x_ref, max_out, idx_out):
    i = pl.program_id(0)
    x = x_ref[...]
    bmax = jnp.max(x, axis=0, keepdims=True)
    rows = lax.broadcasted_iota(jnp.int32, x.shape, 0) + i * BLOCK_ROWS
    bidx = jnp.min(
        jnp.where(x == bmax, rows, jnp.int32(N_ROWS)), axis=0, keepdims=True
    )

    @pl.when(i == 0)
    def _():
        max_out[...] = bmax
        idx_out[...] = bidx

    @pl.when(i > 0)
    def _():
        better = bmax > max_out[...]
        idx_out[...] = jnp.where(better, bidx, idx_out[...])
        max_out[...] = jnp.where(better, bmax, max_out[...])


_maxpool = pl.pallas_call(
    _maxpool_body,
    grid=(NUM_BLOCKS,),
    in_specs=[pl.BlockSpec((BLOCK_ROWS, N_COLS), lambda i: (i, 0))],
    out_specs=[
        pl.BlockSpec((1, N_COLS), lambda i: (0, 0)),
        pl.BlockSpec((1, N_COLS), lambda i: (0, 0)),
    ],
    out_shape=[
        jax.ShapeDtypeStruct((1, N_COLS), jnp.float32),
        jax.ShapeDtypeStruct((1, N_COLS), jnp.int32),
    ],
)


N_SUBCORES = 16
IDX_PER_TILE = N_COLS // N_SUBCORES
ZERO_PER_TILE = N_BINS_PAD // N_SUBCORES
BINS_PER_CORE = N_BINS_PAD // 2


def _hist_body(idx_hbm, out_hbm, idx_row_v, vals_v, zeros_v, hist_sh):
    cid = lax.axis_index("c")
    sid = lax.axis_index("s")
    zeros16 = jnp.zeros((LANES,), jnp.float32)
    for i in range(ZERO_PER_TILE // LANES):
        zeros_v[pl.ds(i * LANES, LANES)] = zeros16
    vals16 = jnp.full((LANES,), INV_TOTAL, jnp.float32)
    for i in range(IDX_PER_TILE // LANES):
        vals_v[pl.ds(i * LANES, LANES)] = vals16
    pltpu.sync_copy(idx_hbm.at[sid], idx_row_v)
    pltpu.sync_copy(zeros_v, hist_sh.at[pl.ds(sid * ZERO_PER_TILE, ZERO_PER_TILE)])
    plsc.subcore_barrier()
    pltpu.sync_copy(vals_v, hist_sh.at[idx_row_v], add=True)
    plsc.subcore_barrier()
    goff = cid * BINS_PER_CORE + sid * BINS_PER_TILE
    out_v = zeros_v.at[pl.ds(0, BINS_PER_TILE)]
    pltpu.sync_copy(hist_sh.at[pl.ds(goff, BINS_PER_TILE)], out_v)
    pltpu.sync_copy(out_v, out_hbm.at[pl.ds(goff, BINS_PER_TILE)])


@functools.cache
def _hist():
    return functools.partial(
        pl.kernel,
        mesh=plsc.VectorSubcoreMesh(core_axis_name="c", subcore_axis_name="s"),
        out_type=jax.ShapeDtypeStruct((N_BINS_PAD,), jnp.float32),
        scratch_types=[
            pltpu.VMEM((IDX_PER_TILE,), jnp.int32),
            pltpu.VMEM((IDX_PER_TILE,), jnp.float32),
            pltpu.VMEM((ZERO_PER_TILE,), jnp.float32),
            pltpu.VMEM_SHARED((N_BINS_PAD,), jnp.float32),
        ],
    )(_hist_body)


@jax.jit
def kernel(features):
    pooled, indices = _maxpool(features)
    hist = _hist()(indices.reshape(N_SUBCORES, IDX_PER_TILE))
    attention_weights = hist[:N_ROWS].reshape(1, N_ROWS)
    return (attention_weights, pooled)


# speedup vs baseline: 1.2616x; 1.0107x over previous
import functools

import jax
import jax.numpy as jnp
from jax import lax
from jax.experimental import pallas as pl
from jax.experimental.pallas import tpu as pltpu
from jax.experimental.pallas import tpu_sc as plsc

N_ROWS = 100000
N_COLS = 512
BLOCK_ROWS = 10000
NUM_BLOCKS = N_ROWS // BLOCK_ROWS

N_TILES = 32
BINS_PER_TILE = 3136
N_BINS_PAD = N_TILES * BINS_PER_TILE
LANES = 16
INV_TOTAL = 1.0 / N_COLS


def _maxpool_body(x_ref, max_out, idx_out):
    i = pl.program_id(0)
    x = x_ref[...]
    bmax = jnp.max(x, axis=0, keepdims=True)
    rows = lax.broadcasted_iota(jnp.int32, x.shape, 0) + i * BLOCK_ROWS
    bidx = jnp.min(
        jnp.where(x == bmax, rows, jnp.int32(N_ROWS)), axis=0, keepdims=True
    )

    @pl.when(i == 0)
    def _():
        max_out[...] = bmax
        idx_out[...] = bidx

    @pl.when(i > 0)
    def _():
        better = bmax > max_out[...]
        idx_out[...] = jnp.where(better, bidx, idx_out[...])
        max_out[...] = jnp.where(better, bmax, max_out[...])


_maxpool = pl.pallas_call(
    _maxpool_body,
    grid=(NUM_BLOCKS,),
    in_specs=[pl.BlockSpec((BLOCK_ROWS, N_COLS), lambda i: (i, 0))],
    out_specs=[
        pl.BlockSpec((1, N_COLS), lambda i: (0, 0)),
        pl.BlockSpec((1, N_COLS), lambda i: (0, 0)),
    ],
    out_shape=[
        jax.ShapeDtypeStruct((1, N_COLS), jnp.float32),
        jax.ShapeDtypeStruct((1, N_COLS), jnp.int32),
    ],
)


N_SUBCORES = 16
IDX_PER_TILE = N_COLS // N_SUBCORES
ZERO_PER_TILE = N_BINS_PAD // N_SUBCORES
BINS_PER_CORE = N_BINS_PAD // 2


def _hist_body(idx_hbm, out_hbm, idx_row_v, vals_v, zeros_v, hist_sh):
    cid = lax.axis_index("c")
    sid = lax.axis_index("s")
    zeros16 = jnp.zeros((LANES,), jnp.float32)
    for i in range(ZERO_PER_TILE // LANES):
        zeros_v[pl.ds(i * LANES, LANES)] = zeros16
    vals16 = jnp.full((LANES,), INV_TOTAL, jnp.float32)
    for i in range(IDX_PER_TILE // LANES):
        vals_v[pl.ds(i * LANES, LANES)] = vals16
    pltpu.sync_copy(idx_hbm.at[sid], idx_row_v)
    pltpu.sync_copy(zeros_v, hist_sh.at[pl.ds(sid * ZERO_PER_TILE, ZERO_PER_TILE)])
    plsc.subcore_barrier()
    pltpu.sync_copy(vals_v, hist_sh.at[idx_row_v], add=True)
    plsc.subcore_barrier()
    goff = cid * BINS_PER_CORE + sid * BINS_PER_TILE
    out_v = zeros_v.at[pl.ds(0, BINS_PER_TILE)]
    pltpu.sync_copy(hist_sh.at[pl.ds(goff, BINS_PER_TILE)], out_v)
    pltpu.sync_copy(out_v, out_hbm.at[pl.ds(goff, BINS_PER_TILE)])


@functools.cache
def _hist():
    return functools.partial(
        pl.kernel,
        mesh=plsc.VectorSubcoreMesh(core_axis_name="c", subcore_axis_name="s"),
        out_type=jax.ShapeDtypeStruct((N_BINS_PAD,), jnp.float32),
        scratch_types=[
            pltpu.VMEM((IDX_PER_TILE,), jnp.int32),
            pltpu.VMEM((IDX_PER_TILE,), jnp.float32),
            pltpu.VMEM((ZERO_PER_TILE,), jnp.float32),
            pltpu.VMEM_SHARED((N_BINS_PAD,), jnp.float32),
        ],
    )(_hist_body)


@jax.jit
def kernel(features):
    pooled, indices = _maxpool(features)
    hist = _hist()(indices.reshape(N_SUBCORES, IDX_PER_TILE))
    attention_weights = hist[:N_ROWS].reshape(1, N_ROWS)
    return (attention_weights, pooled)
